# baseline (speedup 1.0000x reference)
import functools

import jax
import jax.numpy as jnp
from jax import lax
from jax.experimental import pallas as pl
from jax.experimental.pallas import tpu as pltpu
from jax.experimental.pallas import tpu_sc as plsc

N = 10000
E = 320000
D = 128
G = 64

NC, NS = 2, 16
NW = NC * NS
CHUNK = 64
NCHUNK = 160
SUP = 32
NSUP = NCHUNK // SUP
EPT = CHUNK * NCHUNK
EPAD = EPT * NW
TRASH = N
NP = 10240
ZPT = NP // NS

BLK = 2048
GRID = NP // BLK

_SC_CALLS = None


def _get_sc_calls():
    global _SC_CALLS
    if _SC_CALLS is None:
        mesh = plsc.VectorSubcoreMesh(core_axis_name="c", subcore_axis_name="s",
                                      num_cores=NC, num_subcores=NS)
        deg = pl.kernel(
            _deg_body, mesh=mesh,
            out_type=jax.ShapeDtypeStruct((NC, NP, D), jnp.float32),
            scratch_types=[
                pltpu.VMEM((SUP, CHUNK), jnp.int32),
                pltpu.VMEM((CHUNK, D), jnp.float32),
                pltpu.VMEM_SHARED((NP, D), jnp.float32),
            ],
        )
        conv = pl.kernel(
            _conv_body, mesh=mesh,
            out_type=jax.ShapeDtypeStruct((NC, NP, D), jnp.float32),
            scratch_types=[
                pltpu.VMEM((SUP, CHUNK), jnp.int32),
                pltpu.VMEM((SUP, CHUNK), jnp.int32),
                pltpu.VMEM((CHUNK, D), jnp.float32),
                pltpu.VMEM((CHUNK, D), jnp.float32),
                pltpu.VMEM_SHARED((NP, D), jnp.float32),
                pltpu.SemaphoreType.DMA,
                pltpu.SemaphoreType.DMA,
                pltpu.SemaphoreType.DMA,
                pltpu.SemaphoreType.DMA,
            ],
        )
        _SC_CALLS = (deg, conv)
    return _SC_CALLS


def _deg_body(dst_hbm, zeros_hbm, ones_hbm, out_hbm, dstv, onesv, acc):
    c = lax.axis_index("c")
    s = lax.axis_index("s")
    wid = c * NS + s
    pltpu.sync_copy(zeros_hbm.at[pl.ds(s * ZPT, ZPT)], acc.at[pl.ds(s * ZPT, ZPT)])
    pltpu.sync_copy(ones_hbm, onesv)
    plsc.subcore_barrier()

    def sup_step(u, carry):
        pltpu.sync_copy(dst_hbm.at[wid, pl.ds(u * SUP, SUP)], dstv)

        def step(j, carry2):
            pltpu.sync_copy(onesv, acc.at[dstv.at[j]], add=True)
            return carry2

        return lax.fori_loop(0, SUP, step, carry)

    lax.fori_loop(0, NSUP, sup_step, 0)
    plsc.subcore_barrier()
    pltpu.sync_copy(acc.at[pl.ds(s * ZPT, ZPT)], out_hbm.at[c, pl.ds(s * ZPT, ZPT)])


def _conv_body(hs_hbm, src_hbm, dst_hbm, zeros_hbm, out_hbm,
               srcv, dstv, bufa, bufb, acc, sga, sgb, ssa, ssb):
    c = lax.axis_index("c")
    s = lax.axis_index("s")
    wid = c * NS + s
    pltpu.sync_copy(zeros_hbm.at[pl.ds(s * ZPT, ZPT)], acc.at[pl.ds(s * ZPT, ZPT)])
    plsc.subcore_barrier()

    def wait_g(buf, sem):
        pltpu.make_async_copy(hs_hbm.at[srcv.at[0]], buf, sem).wait()

    def wait_s(buf, sem):
        pltpu.make_async_copy(buf, acc.at[dstv.at[0]], sem).wait()

    def sup_step(u, carry):
        pltpu.sync_copy(src_hbm.at[wid, pl.ds(u * SUP, SUP)], srcv)
        pltpu.sync_copy(dst_hbm.at[wid, pl.ds(u * SUP, SUP)], dstv)

        pltpu.async_copy(hs_hbm.at[srcv.at[0]], bufa, sga)
        pltpu.async_copy(hs_hbm.at[srcv.at[1]], bufb, sgb)

        def step(p, carry2):
            j = p * 2
            wait_g(bufa, sga)
            pltpu.async_copy(bufa, acc.at[dstv.at[j]], ssa, add=True)
            wait_g(bufb, sgb)
            pltpu.async_copy(bufb, acc.at[dstv.at[j + 1]], ssb, add=True)

            @pl.when(j + 2 < SUP)
            def _():
                wait_s(bufa, ssa)
                pltpu.async_copy(hs_hbm.at[srcv.at[j + 2]], bufa, sga)

            @pl.when(j + 3 < SUP)
            def _():
                wait_s(bufb, ssb)
                pltpu.async_copy(hs_hbm.at[srcv.at[j + 3]], bufb, sgb)

            return carry2

        lax.fori_loop(0, SUP // 2, step, carry)
        wait_s(bufa, ssa)
        wait_s(bufb, ssb)
        return carry

    lax.fori_loop(0, NSUP, sup_step, 0)
    plsc.subcore_barrier()
    pltpu.sync_copy(acc.at[pl.ds(s * ZPT, ZPT)], out_hbm.at[c, pl.ds(s * ZPT, ZPT)])


def _pre_body(degp, x, w, hs_out, dinv_out):
    deg = degp[0, :, 0:1] + degp[1, :, 0:1] + 1.0
    dinv = lax.rsqrt(deg)
    dinv_out[...] = jnp.broadcast_to(dinv, dinv_out.shape)
    hs_out[...] = jnp.dot(x[...], w[...],
                          preferred_element_type=jnp.float32) * dinv


_pre_call = pl.pallas_call(
    _pre_body,
    grid=(GRID,),
    in_specs=[
        pl.BlockSpec((NC, BLK, D), lambda i: (0, i, 0)),
        pl.BlockSpec((BLK, D), lambda i: (i, 0)),
        pl.BlockSpec((D, D), lambda i: (0, 0)),
    ],
    out_specs=[
        pl.BlockSpec((BLK, D), lambda i: (i, 0)),
        pl.BlockSpec((BLK, 16), lambda i: (i, 0)),
    ],
    out_shape=[
        jax.ShapeDtypeStruct((NP, D), jnp.float32),
        jax.ShapeDtypeStruct((NP, 16), jnp.float32),
    ],
)


def _mid_body(sp, hs, dinv, w, b, out):
    dv = dinv[...][:, 0:1]
    h = jnp.maximum((sp[0] + sp[1] + hs[...]) * dv + b[...], 0.0)
    out[...] = jnp.dot(h, w[...], preferred_element_type=jnp.float32) * dv


_mid_call = pl.pallas_call(
    _mid_body,
    grid=(GRID,),
    in_specs=[
        pl.BlockSpec((NC, BLK, D), lambda i: (0, i, 0)),
        pl.BlockSpec((BLK, D), lambda i: (i, 0)),
        pl.BlockSpec((BLK, 16), lambda i: (i, 0)),
        pl.BlockSpec((D, D), lambda i: (0, 0)),
        pl.BlockSpec((1, D), lambda i: (0, 0)),
    ],
    out_specs=pl.BlockSpec((BLK, D), lambda i: (i, 0)),
    out_shape=jax.ShapeDtypeStruct((NP, D), jnp.float32),
)


def _post_body(sp, hs, dinv, b, batchf, wc, bc, out, sums, counts):
    i = pl.program_id(0)

    @pl.when(i == 0)
    def _():
        sums[...] = jnp.zeros_like(sums)
        counts[...] = jnp.zeros_like(counts)

    dv = dinv[...][:, 0:1]
    h = jnp.maximum((sp[0] + sp[1] + hs[...]) * dv + b[...], 0.0)
    gid = lax.broadcasted_iota(jnp.int32, (BLK, G), 1).astype(jnp.float32)
    onehot = (batchf[...][:, 0:1] == gid).astype(jnp.float32)
    dn = (((0,), (0,)), ((), ()))
    sums[...] += lax.dot_general(onehot, h, dn, preferred_element_type=jnp.float32)
    counts[...] += lax.dot_general(onehot, jnp.ones_like(h), dn,
                                   preferred_element_type=jnp.float32)

    @pl.when(i == pl.num_programs(0) - 1)
    def _():
        g = sums[...] / jnp.maximum(counts[...], 1.0)
        logits = jnp.dot(g, wc[...], preferred_element_type=jnp.float32) + bc[...]
        out[...] = jax.nn.sigmoid(logits)


_post_call = pl.pallas_call(
    _post_body,
    grid=(GRID,),
    in_specs=[
        pl.BlockSpec((NC, BLK, D), lambda i: (0, i, 0)),
        pl.BlockSpec((BLK, D), lambda i: (i, 0)),
        pl.BlockSpec((BLK, 16), lambda i: (i, 0)),
        pl.BlockSpec((1, D), lambda i: (0, 0)),
        pl.BlockSpec((BLK, 16), lambda i: (i, 0)),
        pl.BlockSpec((D, D), lambda i: (0, 0)),
        pl.BlockSpec((1, D), lambda i: (0, 0)),
    ],
    out_specs=pl.BlockSpec((G, D), lambda i: (0, 0)),
    out_shape=jax.ShapeDtypeStruct((G, D), jnp.float32),
    scratch_shapes=[
        pltpu.VMEM((G, D), jnp.float32),
        pltpu.VMEM((G, D), jnp.float32),
    ],
)


def kernel(x, edge_index, batch, W1, b1, W2, b2, Wc, bc):
    src = edge_index[0].astype(jnp.int32)
    dst = edge_index[1].astype(jnp.int32)
    npad = EPAD - E
    src3 = jnp.concatenate([src, jnp.zeros((npad,), jnp.int32)]).reshape(NW, NCHUNK, CHUNK)
    dst3 = jnp.concatenate([dst, jnp.full((npad,), TRASH, jnp.int32)]).reshape(NW, NCHUNK, CHUNK)
    zeros_acc = jnp.zeros((NP, D), jnp.float32)
    ones128 = jnp.ones((CHUNK, D), jnp.float32)
    xp = jnp.concatenate([x, jnp.zeros((NP - N, D), jnp.float32)])
    bpad = jnp.concatenate([batch.astype(jnp.float32),
                            jnp.full((NP - N,), -1.0, jnp.float32)])
    batchf = jnp.broadcast_to(bpad[:, None], (NP, 16))
    b1r = b1.reshape(1, D)
    b2r = b2.reshape(1, D)
    wcp = jnp.pad(Wc, ((0, 0), (0, D - Wc.shape[1])))
    bcp = jnp.broadcast_to(bc.reshape(1, 1), (1, D))

    deg_call, conv_call = _get_sc_calls()
    degp = deg_call(dst3, zeros_acc, ones128)
    hs1, dinv = _pre_call(degp, xp, W1)
    sp1 = conv_call(hs1, src3, dst3, zeros_acc)
    hs2 = _mid_call(sp1, hs1, dinv, W2, b1r)
    sp2 = conv_call(hs2, src3, dst3, zeros_acc)
    outp = _post_call(sp2, hs2, dinv, b2r, batchf, wcp, bcp)
    return outp[:, 0:1]


# speedup vs baseline: 1.0325x; 1.0325x over previous
import functools

import jax
import jax.numpy as jnp
from jax import lax
from jax.experimental import pallas as pl
from jax.experimental.pallas import tpu as pltpu
from jax.experimental.pallas import tpu_sc as plsc

N = 10000
E = 320000
D = 128
G = 64

NC, NS = 2, 16
NW = NC * NS
CHUNK = 64
ROWS = 80
SUPR = 8
NSUP = ROWS // SUPR
EPT = 128 * ROWS
EPAD = EPT * NW
TRASH = N
NP = 10240
ZPT = NP // NS

BLK = 2048
GRID = NP // BLK

_SC_CALLS = None


def _get_sc_calls():
    global _SC_CALLS
    if _SC_CALLS is None:
        mesh = plsc.VectorSubcoreMesh(core_axis_name="c", subcore_axis_name="s",
                                      num_cores=NC, num_subcores=NS)
        deg = pl.kernel(
            _deg_body, mesh=mesh,
            out_type=jax.ShapeDtypeStruct((NC, NP, D), jnp.float32),
            scratch_types=[
                pltpu.VMEM((SUPR, 128), jnp.int32),
                pltpu.VMEM((CHUNK, D), jnp.float32),
                pltpu.VMEM_SHARED((NP, D), jnp.float32),
            ],
        )
        conv = pl.kernel(
            _conv_body, mesh=mesh,
            out_type=jax.ShapeDtypeStruct((NC, NP, D), jnp.float32),
            scratch_types=[
                pltpu.VMEM((SUPR, 128), jnp.int32),
                pltpu.VMEM((SUPR, 128), jnp.int32),
                pltpu.VMEM((CHUNK, D), jnp.float32),
                pltpu.VMEM((CHUNK, D), jnp.float32),
                pltpu.VMEM_SHARED((NP, D), jnp.float32),
                pltpu.SemaphoreType.DMA,
                pltpu.SemaphoreType.DMA,
                pltpu.SemaphoreType.DMA,
                pltpu.SemaphoreType.DMA,
            ],
        )
        _SC_CALLS = (deg, conv)
    return _SC_CALLS


def _deg_body(dst_hbm, zeros_hbm, ones_hbm, out_hbm, dstv, onesv, acc):
    c = lax.axis_index("c")
    s = lax.axis_index("s")
    wid = c * NS + s
    pltpu.sync_copy(zeros_hbm.at[pl.ds(s * ZPT, ZPT)], acc.at[pl.ds(s * ZPT, ZPT)])
    pltpu.sync_copy(ones_hbm, onesv)
    plsc.subcore_barrier()

    def sup_step(u, carry):
        pltpu.sync_copy(dst_hbm.at[wid, pl.ds(u * SUPR, SUPR)], dstv)

        def step(j, carry2):
            pltpu.sync_copy(onesv, acc.at[dstv.at[j // 2, pl.ds((j % 2) * CHUNK, CHUNK)]],
                            add=True)
            return carry2

        return lax.fori_loop(0, 2 * SUPR, step, carry)

    lax.fori_loop(0, NSUP, sup_step, 0)
    plsc.subcore_barrier()
    pltpu.sync_copy(acc.at[pl.ds(s * ZPT, ZPT)], out_hbm.at[c, pl.ds(s * ZPT, ZPT)])


def _conv_body(hs_hbm, src_hbm, dst_hbm, zeros_hbm, out_hbm,
               srcv, dstv, bufa, bufb, acc, sga, sgb, ssa, ssb):
    c = lax.axis_index("c")
    s = lax.axis_index("s")
    wid = c * NS + s
    pltpu.sync_copy(zeros_hbm.at[pl.ds(s * ZPT, ZPT)], acc.at[pl.ds(s * ZPT, ZPT)])
    plsc.subcore_barrier()

    def idx(v, j):
        return v.at[j // 2, pl.ds((j % 2) * CHUNK, CHUNK)]

    def wait_g(buf, sem):
        pltpu.make_async_copy(hs_hbm.at[idx(srcv, 0)], buf, sem).wait()

    def wait_s(buf, sem):
        pltpu.make_async_copy(buf, acc.at[idx(dstv, 0)], sem).wait()

    def sup_step(u, carry):
        pltpu.sync_copy(src_hbm.at[wid, pl.ds(u * SUPR, SUPR)], srcv)
        pltpu.sync_copy(dst_hbm.at[wid, pl.ds(u * SUPR, SUPR)], dstv)

        pltpu.async_copy(hs_hbm.at[idx(srcv, 0)], bufa, sga)
        pltpu.async_copy(hs_hbm.at[idx(srcv, 1)], bufb, sgb)

        def step(p, carry2):
            j = p * 2
            wait_g(bufa, sga)
            pltpu.async_copy(bufa, acc.at[idx(dstv, j)], ssa, add=True)
            wait_g(bufb, sgb)
            pltpu.async_copy(bufb, acc.at[idx(dstv, j + 1)], ssb, add=True)

            @pl.when(j + 2 < 2 * SUPR)
            def _():
                wait_s(bufa, ssa)
                pltpu.async_copy(hs_hbm.at[idx(srcv, j + 2)], bufa, sga)

            @pl.when(j + 3 < 2 * SUPR)
            def _():
                wait_s(bufb, ssb)
                pltpu.async_copy(hs_hbm.at[idx(srcv, j + 3)], bufb, sgb)

            return carry2

        lax.fori_loop(0, SUPR, step, carry)
        wait_s(bufa, ssa)
        wait_s(bufb, ssb)
        return carry

    lax.fori_loop(0, NSUP, sup_step, 0)
    plsc.subcore_barrier()
    pltpu.sync_copy(acc.at[pl.ds(s * ZPT, ZPT)], out_hbm.at[c, pl.ds(s * ZPT, ZPT)])


def _pre_body(degp, x, w, hs_out, dinv_out):
    deg = degp[0, :, 0:1] + degp[1, :, 0:1] + 1.0
    dinv = lax.rsqrt(deg)
    dinv_out[...] = jnp.broadcast_to(dinv, dinv_out.shape)
    hs_out[...] = jnp.dot(x[...], w[...],
                          preferred_element_type=jnp.float32) * dinv


_pre_call = pl.pallas_call(
    _pre_body,
    grid=(GRID,),
    in_specs=[
        pl.BlockSpec((NC, BLK, D), lambda i: (0, i, 0)),
        pl.BlockSpec((BLK, D), lambda i: (i, 0)),
        pl.BlockSpec((D, D), lambda i: (0, 0)),
    ],
    out_specs=[
        pl.BlockSpec((BLK, D), lambda i: (i, 0)),
        pl.BlockSpec((BLK, 16), lambda i: (i, 0)),
    ],
    out_shape=[
        jax.ShapeDtypeStruct((NP, D), jnp.float32),
        jax.ShapeDtypeStruct((NP, 16), jnp.float32),
    ],
)


def _mid_body(sp, hs, dinv, w, b, out):
    dv = dinv[...][:, 0:1]
    h = jnp.maximum((sp[0] + sp[1] + hs[...]) * dv + b[...], 0.0)
    out[...] = jnp.dot(h, w[...], preferred_element_type=jnp.float32) * dv


_mid_call = pl.pallas_call(
    _mid_body,
    grid=(GRID,),
    in_specs=[
        pl.BlockSpec((NC, BLK, D), lambda i: (0, i, 0)),
        pl.BlockSpec((BLK, D), lambda i: (i, 0)),
        pl.BlockSpec((BLK, 16), lambda i: (i, 0)),
        pl.BlockSpec((D, D), lambda i: (0, 0)),
        pl.BlockSpec((1, D), lambda i: (0, 0)),
    ],
    out_specs=pl.BlockSpec((BLK, D), lambda i: (i, 0)),
    out_shape=jax.ShapeDtypeStruct((NP, D), jnp.float32),
)


def _post_body(sp, hs, dinv, b, batchf, wc, bc, out, sums, counts):
    i = pl.program_id(0)

    @pl.when(i == 0)
    def _():
        sums[...] = jnp.zeros_like(sums)
        counts[...] = jnp.zeros_like(counts)

    dv = dinv[...][:, 0:1]
    h = jnp.maximum((sp[0] + sp[1] + hs[...]) * dv + b[...], 0.0)
    gid = lax.broadcasted_iota(jnp.int32, (BLK, G), 1).astype(jnp.float32)
    onehot = (batchf[...][:, 0:1] == gid).astype(jnp.float32)
    dn = (((0,), (0,)), ((), ()))
    sums[...] += lax.dot_general(onehot, h, dn, preferred_element_type=jnp.float32)
    counts[...] += lax.dot_general(onehot, jnp.ones_like(h), dn,
                                   preferred_element_type=jnp.float32)

    @pl.when(i == pl.num_programs(0) - 1)
    def _():
        g = sums[...] / jnp.maximum(counts[...], 1.0)
        logits = jnp.dot(g, wc[...], preferred_element_type=jnp.float32) + bc[...]
        out[...] = jax.nn.sigmoid(logits)


_post_call = pl.pallas_call(
    _post_body,
    grid=(GRID,),
    in_specs=[
        pl.BlockSpec((NC, BLK, D), lambda i: (0, i, 0)),
        pl.BlockSpec((BLK, D), lambda i: (i, 0)),
        pl.BlockSpec((BLK, 16), lambda i: (i, 0)),
        pl.BlockSpec((1, D), lambda i: (0, 0)),
        pl.BlockSpec((BLK, 16), lambda i: (i, 0)),
        pl.BlockSpec((D, D), lambda i: (0, 0)),
        pl.BlockSpec((1, D), lambda i: (0, 0)),
    ],
    out_specs=pl.BlockSpec((G, D), lambda i: (0, 0)),
    out_shape=jax.ShapeDtypeStruct((G, D), jnp.float32),
    scratch_shapes=[
        pltpu.VMEM((G, D), jnp.float32),
        pltpu.VMEM((G, D), jnp.float32),
    ],
)


def kernel(x, edge_index, batch, W1, b1, W2, b2, Wc, bc):
    src = edge_index[0].astype(jnp.int32)
    dst = edge_index[1].astype(jnp.int32)
    npad = EPAD - E
    src3 = jnp.concatenate([src, jnp.zeros((npad,), jnp.int32)]).reshape(NW, ROWS, 128)
    dst3 = jnp.concatenate([dst, jnp.full((npad,), TRASH, jnp.int32)]).reshape(NW, ROWS, 128)
    zeros_acc = jnp.zeros((NP, D), jnp.float32)
    ones128 = jnp.ones((CHUNK, D), jnp.float32)
    xp = jnp.concatenate([x, jnp.zeros((NP - N, D), jnp.float32)])
    bpad = jnp.concatenate([batch.astype(jnp.float32),
                            jnp.full((NP - N,), -1.0, jnp.float32)])
    batchf = jnp.broadcast_to(bpad[:, None], (NP, 16))
    b1r = b1.reshape(1, D)
    b2r = b2.reshape(1, D)
    wcp = jnp.pad(Wc, ((0, 0), (0, D - Wc.shape[1])))
    bcp = jnp.broadcast_to(bc.reshape(1, 1), (1, D))

    deg_call, conv_call = _get_sc_calls()
    degp = deg_call(dst3, zeros_acc, ones128)
    hs1, dinv = _pre_call(degp, xp, W1)
    sp1 = conv_call(hs1, src3, dst3, zeros_acc)
    hs2 = _mid_call(sp1, hs1, dinv, W2, b1r)
    sp2 = conv_call(hs2, src3, dst3, zeros_acc)
    outp = _post_call(sp2, hs2, dinv, b2r, batchf, wcp, bcp)
    return outp[:, 0:1]


# speedup vs baseline: 2.3574x; 2.2832x over previous
import functools

import jax
import jax.numpy as jnp
from jax import lax
from jax.experimental import pallas as pl
from jax.experimental.pallas import tpu as pltpu
from jax.experimental.pallas import tpu_sc as plsc

N = 10000
E = 320000
D = 128
G = 64

NC, NS = 2, 16
NW = NC * NS
CH = 64
EPT = E // NW
BLKE = 1024
NFULL = EPT // BLKE
TAIL = EPT - NFULL * BLKE
NTC = TAIL // CH
REM = TAIL - NTC * CH
NP = 10240
ZPT = NP // NS

BLK = 2048
GRID = NP // BLK

_SC_CALLS = None


def _get_sc_calls():
    global _SC_CALLS
    if _SC_CALLS is None:
        mesh = plsc.VectorSubcoreMesh(core_axis_name="c", subcore_axis_name="s",
                                      num_cores=NC, num_subcores=NS)
        deg = pl.kernel(
            _deg_body, mesh=mesh,
            out_type=jax.ShapeDtypeStruct((NC, NP, D), jnp.float32),
            scratch_types=[
                pltpu.VMEM((BLKE,), jnp.int32),
                pltpu.VMEM((CH, D), jnp.float32),
                pltpu.VMEM_SHARED((NP, D), jnp.float32),
            ],
        )
        conv = pl.kernel(
            _conv_body, mesh=mesh,
            out_type=jax.ShapeDtypeStruct((NC, NP, D), jnp.float32),
            scratch_types=[
                pltpu.VMEM((BLKE,), jnp.int32),
                pltpu.VMEM((BLKE,), jnp.int32),
                pltpu.VMEM((CH, D), jnp.float32),
                pltpu.VMEM((CH, D), jnp.float32),
                pltpu.VMEM_SHARED((NP, D), jnp.float32),
                pltpu.SemaphoreType.DMA,
                pltpu.SemaphoreType.DMA,
                pltpu.SemaphoreType.DMA,
                pltpu.SemaphoreType.DMA,
            ],
        )
        _SC_CALLS = (deg, conv)
    return _SC_CALLS


def _deg_body(dst_hbm, zeros_hbm, ones_hbm, out_hbm, dstv, onesv, acc):
    c = lax.axis_index("c")
    s = lax.axis_index("s")
    wid = c * NS + s
    base = wid * EPT
    pltpu.sync_copy(zeros_hbm.at[pl.ds(s * ZPT, ZPT)], acc.at[pl.ds(s * ZPT, ZPT)])
    pltpu.sync_copy(ones_hbm, onesv)
    plsc.subcore_barrier()

    def sup_step(u, carry):
        pltpu.sync_copy(dst_hbm.at[pl.ds(base + u * BLKE, BLKE)], dstv.at[pl.ds(0, BLKE)])

        def step(j, carry2):
            pltpu.sync_copy(onesv, acc.at[dstv.at[pl.ds(j * CH, CH)]], add=True)
            return carry2

        return lax.fori_loop(0, BLKE // CH, step, carry)

    lax.fori_loop(0, NFULL, sup_step, 0)
    toff = base + NFULL * BLKE
    pltpu.sync_copy(dst_hbm.at[pl.ds(toff, TAIL)], dstv.at[pl.ds(0, TAIL)])

    def tstep(j, carry2):
        pltpu.sync_copy(onesv, acc.at[dstv.at[pl.ds(j * CH, CH)]], add=True)
        return carry2

    lax.fori_loop(0, NTC, tstep, 0)
    pltpu.sync_copy(onesv.at[pl.ds(0, REM)], acc.at[dstv.at[pl.ds(NTC * CH, REM)]], add=True)

    plsc.subcore_barrier()
    pltpu.sync_copy(acc.at[pl.ds(s * ZPT, ZPT)], out_hbm.at[c, pl.ds(s * ZPT, ZPT)])


def _conv_body(hs_hbm, src_hbm, dst_hbm, zeros_hbm, out_hbm,
               srcv, dstv, bufa, bufb, acc, sga, sgb, ssa, ssb):
    c = lax.axis_index("c")
    s = lax.axis_index("s")
    wid = c * NS + s
    base = wid * EPT
    pltpu.sync_copy(zeros_hbm.at[pl.ds(s * ZPT, ZPT)], acc.at[pl.ds(s * ZPT, ZPT)])
    plsc.subcore_barrier()

    def wait_g(buf, sem):
        pltpu.make_async_copy(hs_hbm.at[srcv.at[pl.ds(0, CH)]], buf, sem).wait()

    def wait_s(buf, sem):
        pltpu.make_async_copy(buf, acc.at[dstv.at[pl.ds(0, CH)]], sem).wait()

    def fire_g(buf, sem, j):
        pltpu.async_copy(hs_hbm.at[srcv.at[pl.ds(j * CH, CH)]], buf, sem)

    def fire_s(buf, sem, j):
        pltpu.async_copy(buf, acc.at[dstv.at[pl.ds(j * CH, CH)]], sem, add=True)

    def run_block(nch):
        fire_g(bufa, sga, 0)
        fire_g(bufb, sgb, 1)

        def step(p, carry2):
            j = p * 2
            wait_g(bufa, sga)
            fire_s(bufa, ssa, j)
            wait_g(bufb, sgb)
            fire_s(bufb, ssb, j + 1)

            @pl.when(j + 2 < nch)
            def _():
                wait_s(bufa, ssa)
                fire_g(bufa, sga, j + 2)

            @pl.when(j + 3 < nch)
            def _():
                wait_s(bufb, ssb)
                fire_g(bufb, sgb, j + 3)

            return carry2

        lax.fori_loop(0, nch // 2, step, 0)
        wait_s(bufa, ssa)
        wait_s(bufb, ssb)

    def sup_step(u, carry):
        pltpu.sync_copy(src_hbm.at[pl.ds(base + u * BLKE, BLKE)], srcv.at[pl.ds(0, BLKE)])
        pltpu.sync_copy(dst_hbm.at[pl.ds(base + u * BLKE, BLKE)], dstv.at[pl.ds(0, BLKE)])
        run_block(BLKE // CH)
        return carry

    lax.fori_loop(0, NFULL, sup_step, 0)
    toff = base + NFULL * BLKE
    pltpu.sync_copy(src_hbm.at[pl.ds(toff, TAIL)], srcv.at[pl.ds(0, TAIL)])
    pltpu.sync_copy(dst_hbm.at[pl.ds(toff, TAIL)], dstv.at[pl.ds(0, TAIL)])
    run_block(NTC)
    pltpu.async_copy(hs_hbm.at[srcv.at[pl.ds(NTC * CH, REM)]], bufa.at[pl.ds(0, REM)], sga)
    pltpu.make_async_copy(hs_hbm.at[srcv.at[pl.ds(0, REM)]], bufa.at[pl.ds(0, REM)], sga).wait()
    pltpu.sync_copy(bufa.at[pl.ds(0, REM)], acc.at[dstv.at[pl.ds(NTC * CH, REM)]], add=True)

    plsc.subcore_barrier()
    pltpu.sync_copy(acc.at[pl.ds(s * ZPT, ZPT)], out_hbm.at[c, pl.ds(s * ZPT, ZPT)])


def _pre_body(degp, x, w, hs_out, dinv_out):
    deg = degp[0, :, 0:1] + degp[1, :, 0:1] + 1.0
    dinv = lax.rsqrt(deg)
    dinv_out[...] = jnp.broadcast_to(dinv, dinv_out.shape)
    hs_out[...] = jnp.dot(x[...], w[...],
                          preferred_element_type=jnp.float32) * dinv


_pre_call = pl.pallas_call(
    _pre_body,
    grid=(GRID,),
    in_specs=[
        pl.BlockSpec((NC, BLK, D), lambda i: (0, i, 0)),
        pl.BlockSpec((BLK, D), lambda i: (i, 0)),
        pl.BlockSpec((D, D), lambda i: (0, 0)),
    ],
    out_specs=[
        pl.BlockSpec((BLK, D), lambda i: (i, 0)),
        pl.BlockSpec((BLK, 16), lambda i: (i, 0)),
    ],
    out_shape=[
        jax.ShapeDtypeStruct((NP, D), jnp.float32),
        jax.ShapeDtypeStruct((NP, 16), jnp.float32),
    ],
)


def _mid_body(sp, hs, dinv, w, b, out):
    dv = dinv[...][:, 0:1]
    h = jnp.maximum((sp[0] + sp[1] + hs[...]) * dv + b[...], 0.0)
    out[...] = jnp.dot(h, w[...], preferred_element_type=jnp.float32) * dv


_mid_call = pl.pallas_call(
    _mid_body,
    grid=(GRID,),
    in_specs=[
        pl.BlockSpec((NC, BLK, D), lambda i: (0, i, 0)),
        pl.BlockSpec((BLK, D), lambda i: (i, 0)),
        pl.BlockSpec((BLK, 16), lambda i: (i, 0)),
        pl.BlockSpec((D, D), lambda i: (0, 0)),
        pl.BlockSpec((1, D), lambda i: (0, 0)),
    ],
    out_specs=pl.BlockSpec((BLK, D), lambda i: (i, 0)),
    out_shape=jax.ShapeDtypeStruct((NP, D), jnp.float32),
)


def _post_body(sp, hs, dinv, b, batchf, wc, bc, out, sums, counts):
    i = pl.program_id(0)

    @pl.when(i == 0)
    def _():
        sums[...] = jnp.zeros_like(sums)
        counts[...] = jnp.zeros_like(counts)

    dv = dinv[...][:, 0:1]
    h = jnp.maximum((sp[0] + sp[1] + hs[...]) * dv + b[...], 0.0)
    gid = lax.broadcasted_iota(jnp.int32, (BLK, G), 1).astype(jnp.float32)
    onehot = (batchf[...][:, 0:1] == gid).astype(jnp.float32)
    dn = (((0,), (0,)), ((), ()))
    sums[...] += lax.dot_general(onehot, h, dn, preferred_element_type=jnp.float32)
    counts[...] += lax.dot_general(onehot, jnp.ones_like(h), dn,
                                   preferred_element_type=jnp.float32)

    @pl.when(i == pl.num_programs(0) - 1)
    def _():
        g = sums[...] / jnp.maximum(counts[...], 1.0)
        logits = jnp.dot(g, wc[...], preferred_element_type=jnp.float32) + bc[...]
        out[...] = jax.nn.sigmoid(logits)


_post_call = pl.pallas_call(
    _post_body,
    grid=(GRID,),
    in_specs=[
        pl.BlockSpec((NC, BLK, D), lambda i: (0, i, 0)),
        pl.BlockSpec((BLK, D), lambda i: (i, 0)),
        pl.BlockSpec((BLK, 16), lambda i: (i, 0)),
        pl.BlockSpec((1, D), lambda i: (0, 0)),
        pl.BlockSpec((BLK, 16), lambda i: (i, 0)),
        pl.BlockSpec((D, D), lambda i: (0, 0)),
        pl.BlockSpec((1, D), lambda i: (0, 0)),
    ],
    out_specs=pl.BlockSpec((G, D), lambda i: (0, 0)),
    out_shape=jax.ShapeDtypeStruct((G, D), jnp.float32),
    scratch_shapes=[
        pltpu.VMEM((G, D), jnp.float32),
        pltpu.VMEM((G, D), jnp.float32),
    ],
)


def kernel(x, edge_index, batch, W1, b1, W2, b2, Wc, bc):
    src = edge_index[0].astype(jnp.int32)
    dst = edge_index[1].astype(jnp.int32)
    zeros_acc = jnp.zeros((NP, D), jnp.float32)
    ones128 = jnp.ones((CH, D), jnp.float32)
    xp = jnp.concatenate([x, jnp.zeros((NP - N, D), jnp.float32)])
    bpad = jnp.concatenate([batch.astype(jnp.float32),
                            jnp.full((NP - N,), -1.0, jnp.float32)])
    batchf = jnp.broadcast_to(bpad[:, None], (NP, 16))
    b1r = b1.reshape(1, D)
    b2r = b2.reshape(1, D)
    wcp = jnp.pad(Wc, ((0, 0), (0, D - Wc.shape[1])))
    bcp = jnp.broadcast_to(bc.reshape(1, 1), (1, D))

    deg_call, conv_call = _get_sc_calls()
    degp = deg_call(dst, zeros_acc, ones128)
    hs1, dinv = _pre_call(degp, xp, W1)
    sp1 = conv_call(hs1, src, dst, zeros_acc)
    hs2 = _mid_call(sp1, hs1, dinv, W2, b1r)
    sp2 = conv_call(hs2, src, dst, zeros_acc)
    outp = _post_call(sp2, hs2, dinv, b2r, batchf, wcp, bcp)
    return outp[:, 0:1]


# speedup vs baseline: 2.8865x; 1.2245x over previous
import functools

import jax
import jax.numpy as jnp
from jax import lax
from jax.experimental import pallas as pl
from jax.experimental.pallas import tpu as pltpu
from jax.experimental.pallas import tpu_sc as plsc

N = 10000
E = 320000
D = 128
G = 64

NC, NS = 2, 16
NW = NC * NS
CH = 80
EPT = E // NW
BLKE = 2000
NBLK = EPT // BLKE
CHD = 128
DBLK = 1024
DFULL = EPT // DBLK
DTAIL = EPT - DFULL * DBLK
DNTC = DTAIL // CHD
DREM = DTAIL - DNTC * CHD
NP = 10240
ZPT = NP // NS

BLK = 2048
GRID = NP // BLK

_SC_CALLS = None


def _get_sc_calls():
    global _SC_CALLS
    if _SC_CALLS is None:
        mesh = plsc.VectorSubcoreMesh(core_axis_name="c", subcore_axis_name="s",
                                      num_cores=NC, num_subcores=NS)
        deg = pl.kernel(
            _deg_body, mesh=mesh,
            out_type=jax.ShapeDtypeStruct((NC, NP, D), jnp.float32),
            scratch_types=[
                pltpu.VMEM((DBLK,), jnp.int32),
                pltpu.VMEM((CHD, D), jnp.float32),
                pltpu.VMEM_SHARED((NP, D), jnp.float32),
            ],
        )
        conv = pl.kernel(
            _conv_body, mesh=mesh,
            out_type=jax.ShapeDtypeStruct((NC, NP, D), jnp.float32),
            scratch_types=[
                pltpu.VMEM((BLKE,), jnp.int32),
                pltpu.VMEM((BLKE,), jnp.int32),
                pltpu.VMEM((CH, D), jnp.float32),
                pltpu.VMEM((CH, D), jnp.float32),
                pltpu.VMEM_SHARED((NP, D), jnp.float32),
                pltpu.SemaphoreType.DMA,
                pltpu.SemaphoreType.DMA,
                pltpu.SemaphoreType.DMA,
                pltpu.SemaphoreType.DMA,
            ],
        )
        _SC_CALLS = (deg, conv)
    return _SC_CALLS


def _deg_body(dst_hbm, zeros_hbm, ones_hbm, out_hbm, dstv, onesv, acc):
    c = lax.axis_index("c")
    s = lax.axis_index("s")
    wid = c * NS + s
    base = wid * EPT
    pltpu.sync_copy(zeros_hbm.at[pl.ds(s * ZPT, ZPT)], acc.at[pl.ds(s * ZPT, ZPT)])
    pltpu.sync_copy(ones_hbm, onesv)
    plsc.subcore_barrier()

    def sup_step(u, carry):
        pltpu.sync_copy(dst_hbm.at[pl.ds(base + u * DBLK, DBLK)], dstv.at[pl.ds(0, DBLK)])

        def step(j, carry2):
            pltpu.sync_copy(onesv, acc.at[dstv.at[pl.ds(j * CHD, CHD)]], add=True)
            return carry2

        return lax.fori_loop(0, DBLK // CHD, step, carry)

    lax.fori_loop(0, DFULL, sup_step, 0)
    toff = base + DFULL * DBLK
    pltpu.sync_copy(dst_hbm.at[pl.ds(toff, DTAIL)], dstv.at[pl.ds(0, DTAIL)])

    def tstep(j, carry2):
        pltpu.sync_copy(onesv, acc.at[dstv.at[pl.ds(j * CHD, CHD)]], add=True)
        return carry2

    lax.fori_loop(0, DNTC, tstep, 0)
    pltpu.sync_copy(onesv.at[pl.ds(0, DREM)], acc.at[dstv.at[pl.ds(DNTC * CHD, DREM)]], add=True)

    plsc.subcore_barrier()
    pltpu.sync_copy(acc.at[pl.ds(s * ZPT, ZPT)], out_hbm.at[c, pl.ds(s * ZPT, ZPT)])


def _conv_body(hs_hbm, src_hbm, dst_hbm, zeros_hbm, out_hbm,
               srcv, dstv, bufa, bufb, acc, sga, sgb, ssa, ssb):
    c = lax.axis_index("c")
    s = lax.axis_index("s")
    wid = c * NS + s
    base = wid * EPT
    pltpu.sync_copy(zeros_hbm.at[pl.ds(s * ZPT, ZPT)], acc.at[pl.ds(s * ZPT, ZPT)])
    plsc.subcore_barrier()

    def wait_g(buf, sem):
        pltpu.make_async_copy(hs_hbm.at[srcv.at[pl.ds(0, CH)]], buf, sem).wait()

    def wait_s(buf, sem):
        pltpu.make_async_copy(buf, acc.at[dstv.at[pl.ds(0, CH)]], sem).wait()

    def fire_g(buf, sem, j):
        pltpu.async_copy(hs_hbm.at[srcv.at[pl.ds(j * CH, CH)]], buf, sem)

    def fire_s(buf, sem, j):
        pltpu.async_copy(buf, acc.at[dstv.at[pl.ds(j * CH, CH)]], sem, add=True)

    def run_block(nch):
        fire_g(bufa, sga, 0)

        @pl.when(nch > 1)
        def _():
            fire_g(bufb, sgb, 1)

        def step(p, carry2):
            j = p * 2
            wait_g(bufa, sga)
            fire_s(bufa, ssa, j)

            @pl.when(j + 2 < nch)
            def _():
                wait_s(bufa, ssa)
                fire_g(bufa, sga, j + 2)

            @pl.when(j + 1 < nch)
            def _():
                wait_g(bufb, sgb)
                fire_s(bufb, ssb, j + 1)

                @pl.when(j + 3 < nch)
                def _():
                    wait_s(bufb, ssb)
                    fire_g(bufb, sgb, j + 3)

            return carry2

        lax.fori_loop(0, (nch + 1) // 2, step, 0)
        wait_s(bufa, ssa)

        @pl.when(nch > 1)
        def _():
            wait_s(bufb, ssb)

    def sup_step(u, carry):
        pltpu.sync_copy(src_hbm.at[pl.ds(base + u * BLKE, BLKE)], srcv.at[pl.ds(0, BLKE)])
        pltpu.sync_copy(dst_hbm.at[pl.ds(base + u * BLKE, BLKE)], dstv.at[pl.ds(0, BLKE)])
        run_block(BLKE // CH)
        return carry

    lax.fori_loop(0, NBLK, sup_step, 0)

    plsc.subcore_barrier()
    pltpu.sync_copy(acc.at[pl.ds(s * ZPT, ZPT)], out_hbm.at[c, pl.ds(s * ZPT, ZPT)])


def _pre_body(degp, x, w, hs_out, dinv_out):
    deg = degp[0, :, 0:1] + degp[1, :, 0:1] + 1.0
    dinv = lax.rsqrt(deg)
    dinv_out[...] = jnp.broadcast_to(dinv, dinv_out.shape)
    hs_out[...] = jnp.dot(x[...], w[...],
                          preferred_element_type=jnp.float32) * dinv


_pre_call = pl.pallas_call(
    _pre_body,
    grid=(GRID,),
    in_specs=[
        pl.BlockSpec((NC, BLK, D), lambda i: (0, i, 0)),
        pl.BlockSpec((BLK, D), lambda i: (i, 0)),
        pl.BlockSpec((D, D), lambda i: (0, 0)),
    ],
    out_specs=[
        pl.BlockSpec((BLK, D), lambda i: (i, 0)),
        pl.BlockSpec((BLK, 16), lambda i: (i, 0)),
    ],
    out_shape=[
        jax.ShapeDtypeStruct((NP, D), jnp.float32),
        jax.ShapeDtypeStruct((NP, 16), jnp.float32),
    ],
)


def _mid_body(sp, hs, dinv, w, b, out):
    dv = dinv[...][:, 0:1]
    h = jnp.maximum((sp[0] + sp[1] + hs[...]) * dv + b[...], 0.0)
    out[...] = jnp.dot(h, w[...], preferred_element_type=jnp.float32) * dv


_mid_call = pl.pallas_call(
    _mid_body,
    grid=(GRID,),
    in_specs=[
        pl.BlockSpec((NC, BLK, D), lambda i: (0, i, 0)),
        pl.BlockSpec((BLK, D), lambda i: (i, 0)),
        pl.BlockSpec((BLK, 16), lambda i: (i, 0)),
        pl.BlockSpec((D, D), lambda i: (0, 0)),
        pl.BlockSpec((1, D), lambda i: (0, 0)),
    ],
    out_specs=pl.BlockSpec((BLK, D), lambda i: (i, 0)),
    out_shape=jax.ShapeDtypeStruct((NP, D), jnp.float32),
)


def _post_body(sp, hs, dinv, b, batchf, wc, bc, out, sums, counts):
    i = pl.program_id(0)

    @pl.when(i == 0)
    def _():
        sums[...] = jnp.zeros_like(sums)
        counts[...] = jnp.zeros_like(counts)

    dv = dinv[...][:, 0:1]
    h = jnp.maximum((sp[0] + sp[1] + hs[...]) * dv + b[...], 0.0)
    gid = lax.broadcasted_iota(jnp.int32, (BLK, G), 1).astype(jnp.float32)
    onehot = (batchf[...][:, 0:1] == gid).astype(jnp.float32)
    dn = (((0,), (0,)), ((), ()))
    sums[...] += lax.dot_general(onehot, h, dn, preferred_element_type=jnp.float32)
    counts[...] += lax.dot_general(onehot, jnp.ones_like(h), dn,
                                   preferred_element_type=jnp.float32)

    @pl.when(i == pl.num_programs(0) - 1)
    def _():
        g = sums[...] / jnp.maximum(counts[...], 1.0)
        logits = jnp.dot(g, wc[...], preferred_element_type=jnp.float32) + bc[...]
        out[...] = jax.nn.sigmoid(logits)


_post_call = pl.pallas_call(
    _post_body,
    grid=(GRID,),
    in_specs=[
        pl.BlockSpec((NC, BLK, D), lambda i: (0, i, 0)),
        pl.BlockSpec((BLK, D), lambda i: (i, 0)),
        pl.BlockSpec((BLK, 16), lambda i: (i, 0)),
        pl.BlockSpec((1, D), lambda i: (0, 0)),
        pl.BlockSpec((BLK, 16), lambda i: (i, 0)),
        pl.BlockSpec((D, D), lambda i: (0, 0)),
        pl.BlockSpec((1, D), lambda i: (0, 0)),
    ],
    out_specs=pl.BlockSpec((G, D), lambda i: (0, 0)),
    out_shape=jax.ShapeDtypeStruct((G, D), jnp.float32),
    scratch_shapes=[
        pltpu.VMEM((G, D), jnp.float32),
        pltpu.VMEM((G, D), jnp.float32),
    ],
)


def kernel(x, edge_index, batch, W1, b1, W2, b2, Wc, bc):
    src = edge_index[0].astype(jnp.int32)
    dst = edge_index[1].astype(jnp.int32)
    zeros_acc = jnp.zeros((NP, D), jnp.float32)
    ones128 = jnp.ones((CHD, D), jnp.float32)
    xp = jnp.concatenate([x, jnp.zeros((NP - N, D), jnp.float32)])
    bpad = jnp.concatenate([batch.astype(jnp.float32),
                            jnp.full((NP - N,), -1.0, jnp.float32)])
    batchf = jnp.broadcast_to(bpad[:, None], (NP, 16))
    b1r = b1.reshape(1, D)
    b2r = b2.reshape(1, D)
    wcp = jnp.pad(Wc, ((0, 0), (0, D - Wc.shape[1])))
    bcp = jnp.broadcast_to(bc.reshape(1, 1), (1, D))

    deg_call, conv_call = _get_sc_calls()
    degp = deg_call(dst, zeros_acc, ones128)
    hs1, dinv = _pre_call(degp, xp, W1)
    sp1 = conv_call(hs1, src, dst, zeros_acc)
    hs2 = _mid_call(sp1, hs1, dinv, W2, b1r)
    sp2 = conv_call(hs2, src, dst, zeros_acc)
    outp = _post_call(sp2, hs2, dinv, b2r, batchf, wcp, bcp)
    return outp[:, 0:1]


# speedup vs baseline: 2.8883x; 1.0006x over previous
import functools

import jax
import jax.numpy as jnp
from jax import lax
from jax.experimental import pallas as pl
from jax.experimental.pallas import tpu as pltpu
from jax.experimental.pallas import tpu_sc as plsc

N = 10000
E = 320000
D = 128
G = 64

NC, NS = 2, 16
NW = NC * NS
CH = 80
EPT = E // NW
BLKE = 2000
NBLK = EPT // BLKE
CHD = 128
DBLK = 1024
DFULL = EPT // DBLK
DTAIL = EPT - DFULL * DBLK
DNTC = DTAIL // CHD
DREM = DTAIL - DNTC * CHD
NP = 10240
ZPT = NP // NS

BLK = 2048
GRID = NP // BLK

_SC_CALLS = None


def _get_sc_calls():
    global _SC_CALLS
    if _SC_CALLS is None:
        mesh = plsc.VectorSubcoreMesh(core_axis_name="c", subcore_axis_name="s",
                                      num_cores=NC, num_subcores=NS)
        deg = pl.kernel(
            _deg_body, mesh=mesh,
            out_type=jax.ShapeDtypeStruct((NC, NP, D), jnp.float32),
            scratch_types=[
                pltpu.VMEM((DBLK,), jnp.int32),
                pltpu.VMEM((CHD, D), jnp.float32),
                pltpu.VMEM_SHARED((NP, D), jnp.float32),
                pltpu.SemaphoreType.DMA,
            ],
        )
        conv = pl.kernel(
            _conv_body, mesh=mesh,
            out_type=jax.ShapeDtypeStruct((NC, NP, D), jnp.float32),
            scratch_types=[
                pltpu.VMEM((BLKE,), jnp.int32),
                pltpu.VMEM((BLKE,), jnp.int32),
                pltpu.VMEM((CH, D), jnp.float32),
                pltpu.VMEM((CH, D), jnp.float32),
                pltpu.VMEM_SHARED((NP, D), jnp.float32),
                pltpu.SemaphoreType.DMA,
                pltpu.SemaphoreType.DMA,
                pltpu.SemaphoreType.DMA,
                pltpu.SemaphoreType.DMA,
            ],
        )
        _SC_CALLS = (deg, conv)
    return _SC_CALLS


def _deg_body(dst_hbm, zeros_hbm, ones_hbm, out_hbm, dstv, onesv, acc, sdeg):
    c = lax.axis_index("c")
    s = lax.axis_index("s")
    wid = c * NS + s
    base = wid * EPT
    pltpu.sync_copy(zeros_hbm.at[pl.ds(s * ZPT, ZPT)], acc.at[pl.ds(s * ZPT, ZPT)])
    pltpu.sync_copy(ones_hbm, onesv)
    plsc.subcore_barrier()

    def drain(n):
        def dstep(j, carry2):
            pltpu.make_async_copy(onesv, acc.at[dstv.at[pl.ds(0, CHD)]], sdeg).wait()
            return carry2
        lax.fori_loop(0, n, dstep, 0)

    def sup_step(u, carry):
        pltpu.sync_copy(dst_hbm.at[pl.ds(base + u * DBLK, DBLK)], dstv.at[pl.ds(0, DBLK)])

        def step(j, carry2):
            pltpu.async_copy(onesv, acc.at[dstv.at[pl.ds(j * CHD, CHD)]], sdeg, add=True)
            return carry2

        lax.fori_loop(0, DBLK // CHD, step, carry)
        drain(DBLK // CHD)
        return carry

    lax.fori_loop(0, DFULL, sup_step, 0)
    toff = base + DFULL * DBLK
    pltpu.sync_copy(dst_hbm.at[pl.ds(toff, DTAIL)], dstv.at[pl.ds(0, DTAIL)])

    def tstep(j, carry2):
        pltpu.async_copy(onesv, acc.at[dstv.at[pl.ds(j * CHD, CHD)]], sdeg, add=True)
        return carry2

    lax.fori_loop(0, DNTC, tstep, 0)
    drain(DNTC)
    pltpu.sync_copy(onesv.at[pl.ds(0, DREM)], acc.at[dstv.at[pl.ds(DNTC * CHD, DREM)]], add=True)

    plsc.subcore_barrier()
    pltpu.sync_copy(acc.at[pl.ds(s * ZPT, ZPT)], out_hbm.at[c, pl.ds(s * ZPT, ZPT)])


def _conv_body(hs_hbm, src_hbm, dst_hbm, zeros_hbm, out_hbm,
               srcv, dstv, bufa, bufb, acc, sga, sgb, ssa, ssb):
    c = lax.axis_index("c")
    s = lax.axis_index("s")
    wid = c * NS + s
    base = wid * EPT
    pltpu.sync_copy(zeros_hbm.at[pl.ds(s * ZPT, ZPT)], acc.at[pl.ds(s * ZPT, ZPT)])
    plsc.subcore_barrier()

    def wait_g(buf, sem):
        pltpu.make_async_copy(hs_hbm.at[srcv.at[pl.ds(0, CH)]], buf, sem).wait()

    def wait_s(buf, sem):
        pltpu.make_async_copy(buf, acc.at[dstv.at[pl.ds(0, CH)]], sem).wait()

    def fire_g(buf, sem, j):
        pltpu.async_copy(hs_hbm.at[srcv.at[pl.ds(j * CH, CH)]], buf, sem)

    def fire_s(buf, sem, j):
        pltpu.async_copy(buf, acc.at[dstv.at[pl.ds(j * CH, CH)]], sem, add=True)

    def run_block(nch):
        fire_g(bufa, sga, 0)

        @pl.when(nch > 1)
        def _():
            fire_g(bufb, sgb, 1)

        def step(p, carry2):
            j = p * 2
            wait_g(bufa, sga)
            fire_s(bufa, ssa, j)

            @pl.when(j + 2 < nch)
            def _():
                wait_s(bufa, ssa)
                fire_g(bufa, sga, j + 2)

            @pl.when(j + 1 < nch)
            def _():
                wait_g(bufb, sgb)
                fire_s(bufb, ssb, j + 1)

                @pl.when(j + 3 < nch)
                def _():
                    wait_s(bufb, ssb)
                    fire_g(bufb, sgb, j + 3)

            return carry2

        lax.fori_loop(0, (nch + 1) // 2, step, 0)
        wait_s(bufa, ssa)

        @pl.when(nch > 1)
        def _():
            wait_s(bufb, ssb)

    def sup_step(u, carry):
        pltpu.sync_copy(src_hbm.at[pl.ds(base + u * BLKE, BLKE)], srcv.at[pl.ds(0, BLKE)])
        pltpu.sync_copy(dst_hbm.at[pl.ds(base + u * BLKE, BLKE)], dstv.at[pl.ds(0, BLKE)])
        run_block(BLKE // CH)
        return carry

    lax.fori_loop(0, NBLK, sup_step, 0)

    plsc.subcore_barrier()
    pltpu.sync_copy(acc.at[pl.ds(s * ZPT, ZPT)], out_hbm.at[c, pl.ds(s * ZPT, ZPT)])


def _pre_body(degp, x, w, hs_out, dinv_out):
    deg = degp[0, :, 0:1] + degp[1, :, 0:1] + 1.0
    dinv = lax.rsqrt(deg)
    dinv_out[...] = jnp.broadcast_to(dinv, dinv_out.shape)
    hs_out[...] = jnp.dot(x[...], w[...],
                          preferred_element_type=jnp.float32) * dinv


_pre_call = pl.pallas_call(
    _pre_body,
    grid=(GRID,),
    in_specs=[
        pl.BlockSpec((NC, BLK, D), lambda i: (0, i, 0)),
        pl.BlockSpec((BLK, D), lambda i: (i, 0)),
        pl.BlockSpec((D, D), lambda i: (0, 0)),
    ],
    out_specs=[
        pl.BlockSpec((BLK, D), lambda i: (i, 0)),
        pl.BlockSpec((BLK, 16), lambda i: (i, 0)),
    ],
    out_shape=[
        jax.ShapeDtypeStruct((NP, D), jnp.float32),
        jax.ShapeDtypeStruct((NP, 16), jnp.float32),
    ],
)


def _mid_body(sp, hs, dinv, w, b, out):
    dv = dinv[...][:, 0:1]
    h = jnp.maximum((sp[0] + sp[1] + hs[...]) * dv + b[...], 0.0)
    out[...] = jnp.dot(h, w[...], preferred_element_type=jnp.float32) * dv


_mid_call = pl.pallas_call(
    _mid_body,
    grid=(GRID,),
    in_specs=[
        pl.BlockSpec((NC, BLK, D), lambda i: (0, i, 0)),
        pl.BlockSpec((BLK, D), lambda i: (i, 0)),
        pl.BlockSpec((BLK, 16), lambda i: (i, 0)),
        pl.BlockSpec((D, D), lambda i: (0, 0)),
        pl.BlockSpec((1, D), lambda i: (0, 0)),
    ],
    out_specs=pl.BlockSpec((BLK, D), lambda i: (i, 0)),
    out_shape=jax.ShapeDtypeStruct((NP, D), jnp.float32),
)


def _post_body(sp, hs, dinv, b, batchf, wc, bc, out, sums, counts):
    i = pl.program_id(0)

    @pl.when(i == 0)
    def _():
        sums[...] = jnp.zeros_like(sums)
        counts[...] = jnp.zeros_like(counts)

    dv = dinv[...][:, 0:1]
    h = jnp.maximum((sp[0] + sp[1] + hs[...]) * dv + b[...], 0.0)
    gid = lax.broadcasted_iota(jnp.int32, (BLK, G), 1).astype(jnp.float32)
    onehot = (batchf[...][:, 0:1] == gid).astype(jnp.float32)
    dn = (((0,), (0,)), ((), ()))
    sums[...] += lax.dot_general(onehot, h, dn, preferred_element_type=jnp.float32)
    counts[...] += lax.dot_general(onehot, jnp.ones_like(h), dn,
                                   preferred_element_type=jnp.float32)

    @pl.when(i == pl.num_programs(0) - 1)
    def _():
        g = sums[...] / jnp.maximum(counts[...], 1.0)
        logits = jnp.dot(g, wc[...], preferred_element_type=jnp.float32) + bc[...]
        out[...] = jax.nn.sigmoid(logits)


_post_call = pl.pallas_call(
    _post_body,
    grid=(GRID,),
    in_specs=[
        pl.BlockSpec((NC, BLK, D), lambda i: (0, i, 0)),
        pl.BlockSpec((BLK, D), lambda i: (i, 0)),
        pl.BlockSpec((BLK, 16), lambda i: (i, 0)),
        pl.BlockSpec((1, D), lambda i: (0, 0)),
        pl.BlockSpec((BLK, 16), lambda i: (i, 0)),
        pl.BlockSpec((D, D), lambda i: (0, 0)),
        pl.BlockSpec((1, D), lambda i: (0, 0)),
    ],
    out_specs=pl.BlockSpec((G, D), lambda i: (0, 0)),
    out_shape=jax.ShapeDtypeStruct((G, D), jnp.float32),
    scratch_shapes=[
        pltpu.VMEM((G, D), jnp.float32),
        pltpu.VMEM((G, D), jnp.float32),
    ],
)


def kernel(x, edge_index, batch, W1, b1, W2, b2, Wc, bc):
    src = edge_index[0].astype(jnp.int32)
    dst = edge_index[1].astype(jnp.int32)
    zeros_acc = jnp.zeros((NP, D), jnp.float32)
    ones128 = jnp.ones((CHD, D), jnp.float32)
    xp = jnp.concatenate([x, jnp.zeros((NP - N, D), jnp.float32)])
    bpad = jnp.concatenate([batch.astype(jnp.float32),
                            jnp.full((NP - N,), -1.0, jnp.float32)])
    batchf = jnp.broadcast_to(bpad[:, None], (NP, 16))
    b1r = b1.reshape(1, D)
    b2r = b2.reshape(1, D)
    wcp = jnp.pad(Wc, ((0, 0), (0, D - Wc.shape[1])))
    bcp = jnp.broadcast_to(bc.reshape(1, 1), (1, D))

    deg_call, conv_call = _get_sc_calls()
    degp = deg_call(dst, zeros_acc, ones128)
    hs1, dinv = _pre_call(degp, xp, W1)
    sp1 = conv_call(hs1, src, dst, zeros_acc)
    hs2 = _mid_call(sp1, hs1, dinv, W2, b1r)
    sp2 = conv_call(hs2, src, dst, zeros_acc)
    outp = _post_call(sp2, hs2, dinv, b2r, batchf, wcp, bcp)
    return outp[:, 0:1]
